# baseline (speedup 1.0000x reference)
import functools

import jax
import jax.numpy as jnp
from jax import lax
from jax.experimental import pallas as pl
from jax.experimental.pallas import tpu as pltpu

_TB = 32


def _fused_kernel(x0_ref, x1_ref, x2_ref, x3_ref, x4_ref, x5_ref, x6_ref,
                  x7_ref, a_ref, b_ref, m_ref, g_ref,
                  be_ref, p_ref, o_ref, *, eps, tb, nt, c):
    h = x0_ref.shape[-1]
    hb = tb // 8
    xc = jnp.concatenate(
        [x_ref[b] for x_ref in (x0_ref, x1_ref, x2_ref, x3_ref,
                                x4_ref, x5_ref, x6_ref, x7_ref)
         for b in range(hb)], axis=1)
    zt = lax.dot_general(a_ref[...], xc, (((0,), (0,)), ((), ())),
                         preferred_element_type=jnp.float32)
    zt = zt + b_ref[...]
    st = jnp.concatenate([zt, zt * zt], axis=1)
    mom = jnp.dot(m_ref[...], st, preferred_element_type=jnp.float32)
    n = tb * h
    mean = mom[:, :n]
    var = mom[:, n:] - mean * mean
    y = (zt - mean) * lax.rsqrt(var + eps) * g_ref[...] + be_ref[...]
    ycat = jnp.concatenate(
        [jnp.concatenate([y[:, b * h + t * 128:b * h + (t + 1) * 128]
                          for b in range(tb)], axis=1)
         for t in range(nt)], axis=0)
    o = jnp.dot(p_ref[...], ycat, preferred_element_type=jnp.float32)
    for b in range(tb):
        o_ref[b] = o[:, b * 128:(b + 1) * 128]


def kernel(combine_x, pk_A, pk_b, pk_ln_w, pk_ln_b,
           trade_A, trade_b, trade_ln_w, trade_ln_b):
    eps = 1e-6
    bsz, _, h, w = combine_x.shape
    kp, cp = pk_A.shape
    kt, ct = trade_A.shape
    c = cp + ct
    nt = h // 128

    a = jnp.zeros((w, c), jnp.float32)
    a = a.at[:kp, :cp].set(pk_A.astype(jnp.float32))
    a = a.at[kp:kp + kt, cp:].set(trade_A.astype(jnp.float32))
    bias = jnp.concatenate([pk_b, trade_b]).astype(jnp.float32).reshape(c, 1)
    gamma = jnp.concatenate([pk_ln_w, trade_ln_w]).astype(jnp.float32).reshape(c, 1)
    beta = jnp.concatenate([pk_ln_b, trade_ln_b]).astype(jnp.float32).reshape(c, 1)
    grp = jnp.arange(c) >= cp
    same = grp[:, None] == grp[None, :]
    inv = jnp.where(grp, 1.0 / ct, 1.0 / cp)
    m = jnp.where(same, inv[None, :], 0.0).astype(jnp.float32)
    perm = (jnp.arange(c * nt)[:, None] ==
            ((jnp.arange(c * nt) % c) * nt + jnp.arange(c * nt) // c)[None, :]
            ).astype(jnp.float32)

    xt3 = jnp.transpose(combine_x, (0, 1, 3, 2)).reshape(bsz, w, h)
    tb = _TB
    hb = tb // 8
    grid = (bsz // tb,)
    out = pl.pallas_call(
        functools.partial(_fused_kernel, eps=eps, tb=tb, nt=nt, c=c),
        out_shape=jax.ShapeDtypeStruct((bsz, c * nt, 128), combine_x.dtype),
        grid_spec=pltpu.PrefetchScalarGridSpec(
            num_scalar_prefetch=0,
            grid=grid,
            in_specs=[
                pl.BlockSpec((hb, w, h),
                             functools.partial(lambda k, i: (8 * i + k, 0, 0), k))
                for k in range(8)
            ] + [
                pl.BlockSpec((w, c), lambda i: (0, 0)),
                pl.BlockSpec((c, 1), lambda i: (0, 0)),
                pl.BlockSpec((c, c), lambda i: (0, 0)),
                pl.BlockSpec((c, 1), lambda i: (0, 0)),
                pl.BlockSpec((c, 1), lambda i: (0, 0)),
                pl.BlockSpec((c * nt, c * nt), lambda i: (0, 0)),
            ],
            out_specs=pl.BlockSpec((tb, c * nt, 128), lambda i: (i, 0, 0)),
        ),
        compiler_params=pltpu.CompilerParams(
            dimension_semantics=("arbitrary",),
            vmem_limit_bytes=100 * 1024 * 1024),
        cost_estimate=pl.CostEstimate(
            flops=int(2 * bsz * h * (w * c + 2 * c * c + c * c * nt) + 10 * bsz * h * c),
            transcendentals=int(bsz * h),
            bytes_accessed=int(4 * bsz * h * (w + c))),
    )(*([xt3] * 8), a, bias, m, gamma, beta, perm)
    return out.reshape(bsz, c, h, 1)


# speedup vs baseline: 1.0046x; 1.0046x over previous
import functools

import jax
import jax.numpy as jnp
from jax import lax
from jax.experimental import pallas as pl
from jax.experimental.pallas import tpu as pltpu

_TB = 32


def _fused_kernel(x0_ref, x1_ref, x2_ref, x3_ref, a_ref, b_ref, m_ref, g_ref,
                  be_ref, p_ref, o_ref, *, eps, tb, nt, c):
    h = x0_ref.shape[-1]
    hb = tb // 4
    xc = jnp.concatenate(
        [x_ref[b] for x_ref in (x0_ref, x1_ref, x2_ref, x3_ref)
         for b in range(hb)], axis=1)
    zt = lax.dot_general(a_ref[...], xc, (((0,), (0,)), ((), ())),
                         preferred_element_type=jnp.float32)
    zt = zt + b_ref[...]
    st = jnp.concatenate([zt, zt * zt], axis=1)
    mom = jnp.dot(m_ref[...], st, preferred_element_type=jnp.float32)
    n = tb * h
    mean = mom[:, :n]
    var = mom[:, n:] - mean * mean
    y = (zt - mean) * lax.rsqrt(var + eps) * g_ref[...] + be_ref[...]
    ycat = jnp.concatenate(
        [jnp.concatenate([y[:, b * h + t * 128:b * h + (t + 1) * 128]
                          for b in range(tb)], axis=1)
         for t in range(nt)], axis=0)
    o = jnp.dot(p_ref[...], ycat, preferred_element_type=jnp.float32)
    for b in range(tb):
        o_ref[b] = o[:, b * 128:(b + 1) * 128]


def kernel(combine_x, pk_A, pk_b, pk_ln_w, pk_ln_b,
           trade_A, trade_b, trade_ln_w, trade_ln_b):
    eps = 1e-6
    bsz, _, h, w = combine_x.shape
    kp, cp = pk_A.shape
    kt, ct = trade_A.shape
    c = cp + ct
    nt = h // 128

    a = jnp.zeros((w, c), jnp.float32)
    a = a.at[:kp, :cp].set(pk_A.astype(jnp.float32))
    a = a.at[kp:kp + kt, cp:].set(trade_A.astype(jnp.float32))
    bias = jnp.concatenate([pk_b, trade_b]).astype(jnp.float32).reshape(c, 1)
    gamma = jnp.concatenate([pk_ln_w, trade_ln_w]).astype(jnp.float32).reshape(c, 1)
    beta = jnp.concatenate([pk_ln_b, trade_ln_b]).astype(jnp.float32).reshape(c, 1)
    grp = jnp.arange(c) >= cp
    same = grp[:, None] == grp[None, :]
    inv = jnp.where(grp, 1.0 / ct, 1.0 / cp)
    m = jnp.where(same, inv[None, :], 0.0).astype(jnp.float32)
    perm = (jnp.arange(c * nt)[:, None] ==
            ((jnp.arange(c * nt) % c) * nt + jnp.arange(c * nt) // c)[None, :]
            ).astype(jnp.float32)

    xt3 = jnp.transpose(combine_x, (0, 1, 3, 2)).reshape(bsz, w, h)
    tb = _TB
    hb = tb // 4
    grid = (bsz // tb,)
    out = pl.pallas_call(
        functools.partial(_fused_kernel, eps=eps, tb=tb, nt=nt, c=c),
        out_shape=jax.ShapeDtypeStruct((bsz, c * nt, 128), combine_x.dtype),
        grid_spec=pltpu.PrefetchScalarGridSpec(
            num_scalar_prefetch=0,
            grid=grid,
            in_specs=[
                pl.BlockSpec((hb, w, h),
                             functools.partial(lambda k, i: (4 * i + k, 0, 0), k))
                for k in range(4)
            ] + [
                pl.BlockSpec((w, c), lambda i: (0, 0)),
                pl.BlockSpec((c, 1), lambda i: (0, 0)),
                pl.BlockSpec((c, c), lambda i: (0, 0)),
                pl.BlockSpec((c, 1), lambda i: (0, 0)),
                pl.BlockSpec((c, 1), lambda i: (0, 0)),
                pl.BlockSpec((c * nt, c * nt), lambda i: (0, 0)),
            ],
            out_specs=pl.BlockSpec((tb, c * nt, 128), lambda i: (i, 0, 0)),
        ),
        compiler_params=pltpu.CompilerParams(
            dimension_semantics=("arbitrary",),
            vmem_limit_bytes=100 * 1024 * 1024),
        cost_estimate=pl.CostEstimate(
            flops=int(2 * bsz * h * (w * c + 2 * c * c + c * c * nt) + 10 * bsz * h * c),
            transcendentals=int(bsz * h),
            bytes_accessed=int(4 * bsz * h * (w + c))),
    )(*([xt3] * 4), a, bias, m, gamma, beta, perm)
    return out.reshape(bsz, c, h, 1)
